# baseline (speedup 1.0000x reference)
import functools

import jax
import jax.numpy as jnp
from jax import lax
from jax.experimental import pallas as pl
from jax.experimental.pallas import tpu as pltpu
from jax.experimental.pallas import tpu_sc as plsc


def _format_call(Et, Etail_flat):
    D, V = Et.shape
    n_tail, = Etail_flat.shape
    V_full = V - n_tail // D
    n_full = V_full // 128
    n_even = n_full - (n_full % 32)
    mesh = plsc.VectorSubcoreMesh(core_axis_name="c", subcore_axis_name="s")

    @functools.partial(
        pl.kernel,
        mesh=mesh,
        out_type=jax.ShapeDtypeStruct((V * D,), jnp.float32),
        scratch_types=[
            pltpu.VMEM((D, 128), jnp.float32),
            pltpu.VMEM((D, 128), jnp.float32),
            pltpu.VMEM((128 * D,), jnp.float32),
            pltpu.VMEM((128 * D,), jnp.float32),
            pltpu.VMEM((n_tail,), jnp.float32),
            pltpu.SemaphoreType.DMA,
            pltpu.SemaphoreType.DMA,
            pltpu.SemaphoreType.DMA,
            pltpu.SemaphoreType.DMA,
        ],
        compiler_params=pltpu.CompilerParams(needs_layout_passes=False),
    )
    def format_kernel(et_hbm, tail_hbm, out_hbm, sbuf0, sbuf1, rbuf0, rbuf1,
                      tbuf, isem0, isem1, osem0, osem1):
        sbufs, rbufs = (sbuf0, sbuf1), (rbuf0, rbuf1)
        isems, osems = (isem0, isem1), (osem0, osem1)
        nc = lax.axis_size("c")
        wid = lax.axis_index("s") * nc + lax.axis_index("c")
        pos = [(lax.iota(jnp.int32, 16) + 16 * k) * D for k in range(8)]
        per_w = n_even // 32
        base = wid * per_w

        def start_in(b, s):
            pltpu.async_copy(et_hbm.at[:, pl.ds(s * 128, 128)], sbufs[b],
                             isems[b])

        def wait_in(b):
            pltpu.make_async_copy(et_hbm.at[:, pl.ds(0, 128)], sbufs[b],
                                  isems[b]).wait()

        def transpose(b):
            @plsc.parallel_loop(0, D, unroll=8)
            def d_loop(d):
                dvec = jnp.broadcast_to(d, (16,)).astype(jnp.int32)
                for k in range(8):
                    vals = sbufs[b][d, pl.ds(16 * k, 16)]
                    plsc.store_scatter(rbufs[b], [pos[k] + dvec], vals)

        def start_out(b, s):
            pltpu.async_copy(rbufs[b],
                             out_hbm.at[pl.ds(s * 128 * D, 128 * D)],
                             osems[b])

        def wait_out(b):
            pltpu.make_async_copy(rbufs[b], out_hbm.at[pl.ds(0, 128 * D)],
                                  osems[b]).wait()

        for b in range(2):
            start_in(b, base + b)

        def group_safe(g, carry):
            for b in range(2):
                s = base + 2 * g + b
                wait_in(b)

                @pl.when(g > 0)
                def _():
                    wait_out(b)

                transpose(b)
                start_out(b, s)

                @pl.when(2 * g + b + 2 < per_w)
                def _():
                    start_in(b, s + 2)
            return carry

        lax.fori_loop(0, per_w // 2, group_safe, 0)
        for b in range(2):
            wait_out(b)

        for r in range(n_full - n_even):
            @pl.when(wid == r)
            def _(s=n_even + r):
                start_in(0, s)
                wait_in(0)
                transpose(0)
                start_out(0, s)
                wait_out(0)

        @pl.when(wid == 32 - 1)
        def _():
            pltpu.sync_copy(tail_hbm, tbuf)
            pltpu.sync_copy(tbuf, out_hbm.at[pl.ds(V_full * D, n_tail)])

    return format_kernel(Et, Etail_flat)


def _gather_call(idx_flat, E, n_workers, rows_per_worker, chunk, nbuf):
    Bf, = idx_flat.shape
    V, D = E.shape
    n_chunks = rows_per_worker // chunk
    n_groups = n_chunks // nbuf
    assert n_chunks % nbuf == 0

    mesh = plsc.VectorSubcoreMesh(core_axis_name="c", subcore_axis_name="s")

    @functools.partial(
        pl.kernel,
        mesh=mesh,
        out_type=jax.ShapeDtypeStruct((Bf, 2 * D), jnp.float32),
        scratch_types=(
            [pltpu.VMEM((rows_per_worker,), jnp.int32)]
            + [pltpu.VMEM((chunk, D), jnp.float32) for _ in range(nbuf)]
            + [pltpu.SemaphoreType.DMA for _ in range(2 * nbuf)]
        ),
        compiler_params=pltpu.CompilerParams(use_tc_tiling_on_sc=False),
    )
    def gather_kernel(idx_hbm, table_hbm, out_hbm, idx_v, *bufs_and_sems):
        bufs = bufs_and_sems[:nbuf]
        gsems = bufs_and_sems[nbuf:2 * nbuf]
        osems = bufs_and_sems[2 * nbuf:]
        nc = lax.axis_size("c")
        wid = lax.axis_index("s") * nc + lax.axis_index("c")
        base = wid * rows_per_worker
        pltpu.sync_copy(idx_hbm.at[pl.ds(base, rows_per_worker)], idx_v)

        def start_gather(b, j):
            pltpu.async_copy(
                table_hbm.at[idx_v.at[pl.ds(j * chunk, chunk)]], bufs[b],
                gsems[b])

        def wait_gather(b):
            pltpu.make_async_copy(
                table_hbm.at[pl.ds(0, chunk)], bufs[b], gsems[b]).wait()

        def start_out(b, j):
            pltpu.async_copy(
                bufs[b],
                out_hbm.at[pl.ds(base + j * chunk, chunk), pl.ds(0, D)],
                osems[b])

        def wait_out(b):
            pltpu.make_async_copy(
                bufs[b], out_hbm.at[pl.ds(base, chunk), pl.ds(0, D)],
                osems[b]).wait()

        for b in range(nbuf):
            start_gather(b, b)

        def group(g, carry):
            j_prev = (g - 1) * nbuf
            j_next = g * nbuf
            for b in range(nbuf):
                wait_gather(b)
                start_out(b, j_prev + b)
            for b in range(nbuf):
                wait_out(b)
                start_gather(b, j_next + b)
            return carry

        lax.fori_loop(1, n_groups, group, 0)

        j_last = (n_groups - 1) * nbuf
        for b in range(nbuf):
            wait_gather(b)
            start_out(b, j_last + b)
        for b in range(nbuf):
            wait_out(b)

    return gather_kernel(idx_flat, E)


def kernel(x, E):
    B, H = x.shape
    V, D = E.shape
    Bf = B * H
    info = plsc.get_sparse_core_info()
    n_workers = info.num_cores * info.num_subcores
    rows_per_worker = Bf // n_workers
    V_full = (V // 128) * 128
    table = _format_call(E.T, E[V_full:].reshape(-1)).reshape(V, D)
    out = _gather_call(x.reshape(Bf).astype(jnp.int32), table,
                       n_workers, rows_per_worker, chunk=128, nbuf=5)
    return out.reshape(B, H, 2 * D)[:, :, :D]


# speedup vs baseline: 1.2859x; 1.2859x over previous
import functools

import jax
import jax.numpy as jnp
from jax import lax
from jax.experimental import pallas as pl
from jax.experimental.pallas import tpu as pltpu
from jax.experimental.pallas import tpu_sc as plsc


def _gather_call(idx_flat, E, n_workers, rows_per_worker, chunk, nbuf):
    Bf, = idx_flat.shape
    V, D = E.shape
    n_chunks = rows_per_worker // chunk
    n_groups = n_chunks // nbuf
    assert n_chunks % nbuf == 0

    mesh = plsc.VectorSubcoreMesh(core_axis_name="c", subcore_axis_name="s")

    @functools.partial(
        pl.kernel,
        mesh=mesh,
        out_type=jax.ShapeDtypeStruct((Bf, 2 * D), jnp.float32),
        scratch_types=(
            [pltpu.VMEM((rows_per_worker,), jnp.int32)]
            + [pltpu.VMEM((chunk, D), jnp.float32) for _ in range(nbuf)]
            + [pltpu.SemaphoreType.DMA for _ in range(2 * nbuf)]
        ),
        compiler_params=pltpu.CompilerParams(use_tc_tiling_on_sc=False),
    )
    def gather_kernel(idx_hbm, table_hbm, out_hbm, idx_v, *bufs_and_sems):
        bufs = bufs_and_sems[:nbuf]
        gsems = bufs_and_sems[nbuf:2 * nbuf]
        osems = bufs_and_sems[2 * nbuf:]
        nc = lax.axis_size("c")
        wid = lax.axis_index("s") * nc + lax.axis_index("c")
        base = wid * rows_per_worker
        pltpu.sync_copy(idx_hbm.at[pl.ds(base, rows_per_worker)], idx_v)

        def start_gather(b, j):
            pltpu.async_copy(
                table_hbm.at[idx_v.at[pl.ds(j * chunk, chunk)]], bufs[b],
                gsems[b])

        def wait_gather(b):
            pltpu.make_async_copy(
                table_hbm.at[pl.ds(0, chunk)], bufs[b], gsems[b]).wait()

        def start_out(b, j):
            pltpu.async_copy(
                bufs[b],
                out_hbm.at[pl.ds(base + j * chunk, chunk), pl.ds(0, D)],
                osems[b])

        def wait_out(b):
            pltpu.make_async_copy(
                bufs[b], out_hbm.at[pl.ds(base, chunk), pl.ds(0, D)],
                osems[b]).wait()

        for b in range(nbuf):
            start_gather(b, b)

        def group(g, carry):
            j_prev = (g - 1) * nbuf
            j_next = g * nbuf
            for b in range(nbuf):
                wait_gather(b)
                start_out(b, j_prev + b)
            for b in range(nbuf):
                wait_out(b)
                start_gather(b, j_next + b)
            return carry

        lax.fori_loop(1, n_groups, group, 0)

        j_last = (n_groups - 1) * nbuf
        for b in range(nbuf):
            wait_gather(b)
            start_out(b, j_last + b)
        for b in range(nbuf):
            wait_out(b)

    return gather_kernel(idx_flat, E)


def kernel(x, E):
    B, H = x.shape
    V, D = E.shape
    Bf = B * H
    info = plsc.get_sparse_core_info()
    n_workers = info.num_cores * info.num_subcores
    rows_per_worker = Bf // n_workers
    out = _gather_call(x.reshape(Bf).astype(jnp.int32), E,
                       n_workers, rows_per_worker, chunk=128, nbuf=5)
    return out.reshape(B, H, 2 * D)[:, :, :D]
